# baseline (speedup 1.0000x reference)
import jax
import jax.numpy as jnp
from jax.experimental import pallas as pl
from jax.experimental.pallas import tpu as pltpu


def _slice_body(x_ref, o_ref):
    o_ref[...] = x_ref[:, :, :2]


def kernel(x):
    B = 64
    n, s, _ = x.shape
    return pl.pallas_call(
        _slice_body,
        grid=(n // B,),
        in_specs=[pl.BlockSpec((B, s, 128), lambda i: (i, 0, 0))],
        out_specs=pl.BlockSpec((B, s, 2), lambda i: (i, 0, 0)),
        out_shape=jax.ShapeDtypeStruct((n, s, 2), x.dtype),
    )(x)
